# baseline (speedup 1.0000x reference)
import functools

import jax
import jax.numpy as jnp
from jax import lax
from jax.experimental import pallas as pl
from jax.experimental.pallas import tpu as pltpu
from jax.experimental.pallas import tpu_sc as plsc

NC = 2
NS = 16
NW = NC * NS
LANES = 16

VOCAB = 1000000
DIM = 64
OUT = 2
B = 4096
L = 200

HALF = L // 2
ROWS_PER_W = B // NW
CHUNKS_PER_W = 2 * ROWS_PER_W

_PROJ_BLK = 8192


_PROJ_SUB = _PROJ_BLK // 8
_N_BLOCKS = pl.cdiv(VOCAB, _PROJ_BLK)
_P_ROWS = _N_BLOCKS * _PROJ_SUB
_P_CHUNKS = _P_ROWS * 8


def _proj_body(t_ref, w_ref, o_ref):
    parts = []
    for q in range(8):
        sub = t_ref[q * _PROJ_SUB:(q + 1) * _PROJ_SUB, :]
        parts.append(lax.dot_general(
            sub, w_ref[...],
            dimension_numbers=(((1,), (1,)), ((), ())),
            preferred_element_type=jnp.float32,
        ))
    packed = jnp.concatenate(parts, axis=1)
    o_ref[...] = packed.reshape(_PROJ_SUB * 8 * LANES)


def _project(table, w16):
    return pl.pallas_call(
        _proj_body,
        grid=(_N_BLOCKS,),
        in_specs=[
            pl.BlockSpec((_PROJ_BLK, DIM), lambda i: (i, 0)),
            pl.BlockSpec((LANES, DIM), lambda i: (0, 0)),
        ],
        out_specs=pl.BlockSpec((_PROJ_SUB * 8 * LANES,), lambda i: (i,)),
        out_shape=jax.ShapeDtypeStruct((_P_ROWS * 8 * LANES,), jnp.float32),
    )(table, w16)


def _pool_body(idx_hbm, p_hbm, bias_hbm, out_hbm, idx_v, rows_v, res_v, bias_v, sem):
    wid = lax.axis_index("s") * NC + lax.axis_index("c")
    p_lin = p_hbm
    pltpu.sync_copy(idx_hbm.at[wid], idx_v)
    pltpu.sync_copy(bias_hbm, bias_v)
    inv_l = jnp.float32(1.0 / L)
    bias = bias_v[...]

    def row_body(r, _):
        accs = (
            jnp.zeros((LANES,), jnp.float32),
            jnp.zeros((LANES,), jnp.float32),
            jnp.zeros((LANES,), jnp.float32),
            jnp.zeros((LANES,), jnp.float32),
        )
        for h in range(2):
            j = r * 2 + h
            pltpu.async_copy(p_lin.at[idx_v.at[j]], rows_v, sem).wait()

            def red(i, a):
                a0, a1, a2, a3 = a
                a0 = a0 + rows_v[4 * i, :]
                a1 = a1 + rows_v[4 * i + 1, :]
                a2 = a2 + rows_v[4 * i + 2, :]
                a3 = a3 + rows_v[4 * i + 3, :]
                return (a0, a1, a2, a3)

            accs = lax.fori_loop(0, HALF // 4, red, accs)
        total = (accs[0] + accs[1]) + (accs[2] + accs[3])
        res_v[r, :] = total * inv_l + bias
        return 0

    lax.fori_loop(0, ROWS_PER_W, row_body, 0)
    pltpu.sync_copy(res_v, out_hbm.at[pl.ds(wid * ROWS_PER_W, ROWS_PER_W)])


@functools.cache
def _pool():
    return pl.kernel(
        _pool_body,
        out_type=jax.ShapeDtypeStruct((B, LANES), jnp.float32),
        mesh=plsc.VectorSubcoreMesh(
            core_axis_name="c", subcore_axis_name="s",
            num_cores=NC, num_subcores=NS,
        ),
        scratch_types=[
            pltpu.VMEM((CHUNKS_PER_W, HALF), jnp.int32),
            pltpu.VMEM((HALF, LANES), jnp.float32),
            pltpu.VMEM((ROWS_PER_W, LANES), jnp.float32),
            pltpu.VMEM((LANES,), jnp.float32),
            pltpu.SemaphoreType.DMA,
        ],
        compiler_params=pltpu.CompilerParams(use_tc_tiling_on_sc=False),
    )


@jax.jit
def kernel(text, table, W, b):
    w16 = jnp.zeros((LANES, DIM), jnp.float32).at[:OUT].set(W)
    bias16 = jnp.zeros((LANES,), jnp.float32).at[:OUT].set(b)
    p = _project(table, w16).reshape(_P_CHUNKS, LANES)
    v = text.astype(jnp.int32)
    remap = (v & -8192) | ((v & 1023) << 3) | ((v >> 10) & 7)
    idx = remap.reshape(NW, CHUNKS_PER_W, HALF)
    pooled = _pool()(idx, p, bias16)
    return pooled[:, :OUT]


# speedup vs baseline: 1.5810x; 1.5810x over previous
import functools

import jax
import jax.numpy as jnp
from jax import lax
from jax.experimental import pallas as pl
from jax.experimental.pallas import tpu as pltpu
from jax.experimental.pallas import tpu_sc as plsc

NC = 2
NS = 16
NW = NC * NS
LANES = 16

VOCAB = 1000000
DIM = 64
OUT = 2
B = 4096
L = 200

HALF = L // 2
ROWS_PER_W = B // NW
CHUNKS_PER_W = 2 * ROWS_PER_W

_PROJ_BLK = 8192


_PROJ_SUB = _PROJ_BLK // 8
_N_BLOCKS = pl.cdiv(VOCAB, _PROJ_BLK)
_P_ROWS = _N_BLOCKS * _PROJ_SUB
_P_CHUNKS = _P_ROWS * 8


def _proj_body(tt_ref, w_ref, o_ref):
    parts = []
    for q in range(8):
        sub = tt_ref[:, q * _PROJ_SUB:(q + 1) * _PROJ_SUB]
        parts.append(lax.dot_general(
            sub, w_ref[...],
            dimension_numbers=(((0,), (1,)), ((), ())),
            preferred_element_type=jnp.float32,
        ))
    packed = jnp.concatenate(parts, axis=1)
    o_ref[...] = packed.reshape(_PROJ_SUB * 8 * LANES)


def _project(table_t, w16):
    return pl.pallas_call(
        _proj_body,
        grid=(_N_BLOCKS,),
        in_specs=[
            pl.BlockSpec((DIM, _PROJ_BLK), lambda i: (0, i)),
            pl.BlockSpec((LANES, DIM), lambda i: (0, 0)),
        ],
        out_specs=pl.BlockSpec((_PROJ_SUB * 8 * LANES,), lambda i: (i,)),
        out_shape=jax.ShapeDtypeStruct((_P_ROWS * 8 * LANES,), jnp.float32),
    )(table_t, w16)


def _pool_body(idx_hbm, p_hbm, bias_hbm, out_hbm, idx_v, rows_v, res_v, bias_v, sem):
    wid = lax.axis_index("s") * NC + lax.axis_index("c")
    p_lin = p_hbm
    pltpu.sync_copy(idx_hbm.at[wid], idx_v)
    pltpu.sync_copy(bias_hbm, bias_v)
    inv_l = jnp.float32(1.0 / L)
    bias = bias_v[...]

    def row_body(r, _):
        accs = (
            jnp.zeros((LANES,), jnp.float32),
            jnp.zeros((LANES,), jnp.float32),
            jnp.zeros((LANES,), jnp.float32),
            jnp.zeros((LANES,), jnp.float32),
        )
        for h in range(2):
            j = r * 2 + h
            pltpu.async_copy(p_lin.at[idx_v.at[j]], rows_v, sem).wait()

            def red(i, a):
                a0, a1, a2, a3 = a
                a0 = a0 + rows_v[4 * i, :]
                a1 = a1 + rows_v[4 * i + 1, :]
                a2 = a2 + rows_v[4 * i + 2, :]
                a3 = a3 + rows_v[4 * i + 3, :]
                return (a0, a1, a2, a3)

            accs = lax.fori_loop(0, HALF // 4, red, accs)
        total = (accs[0] + accs[1]) + (accs[2] + accs[3])
        res_v[r, :] = total * inv_l + bias
        return 0

    lax.fori_loop(0, ROWS_PER_W, row_body, 0)
    pltpu.sync_copy(res_v, out_hbm.at[pl.ds(wid * ROWS_PER_W, ROWS_PER_W)])


@functools.cache
def _pool():
    return pl.kernel(
        _pool_body,
        out_type=jax.ShapeDtypeStruct((B, LANES), jnp.float32),
        mesh=plsc.VectorSubcoreMesh(
            core_axis_name="c", subcore_axis_name="s",
            num_cores=NC, num_subcores=NS,
        ),
        scratch_types=[
            pltpu.VMEM((CHUNKS_PER_W, HALF), jnp.int32),
            pltpu.VMEM((HALF, LANES), jnp.float32),
            pltpu.VMEM((ROWS_PER_W, LANES), jnp.float32),
            pltpu.VMEM((LANES,), jnp.float32),
            pltpu.SemaphoreType.DMA,
        ],
        compiler_params=pltpu.CompilerParams(use_tc_tiling_on_sc=False),
    )


@jax.jit
def kernel(text, table, W, b):
    w16 = jnp.zeros((LANES, DIM), jnp.float32).at[:OUT].set(W)
    bias16 = jnp.zeros((LANES,), jnp.float32).at[:OUT].set(b)
    p = _project(table.T, w16).reshape(_P_CHUNKS, LANES)
    v = text.astype(jnp.int32)
    remap = (v & -8192) | ((v & 1023) << 3) | ((v >> 10) & 7)
    idx = remap.reshape(NW, CHUNKS_PER_W, HALF)
    pooled = _pool()(idx, p, bias16)
    return pooled[:, :OUT]


# speedup vs baseline: 2.3602x; 1.4929x over previous
import functools

import jax
import jax.numpy as jnp
from jax import lax
from jax.experimental import pallas as pl
from jax.experimental.pallas import tpu as pltpu
from jax.experimental.pallas import tpu_sc as plsc

NC = 2
NS = 16
NW = NC * NS
LANES = 16

VOCAB = 1000000
DIM = 64
OUT = 2
B = 4096
L = 200

HALF = L // 2
ROWS_PER_W = B // NW
CHUNKS_PER_W = 2 * ROWS_PER_W

_PROJ_BLK = 8192


_PROJ_SUB = _PROJ_BLK // 8
_N_BLOCKS = pl.cdiv(VOCAB, _PROJ_BLK)
_P_ROWS = _N_BLOCKS * _PROJ_SUB
_P_CHUNKS = _P_ROWS * 8


def _proj_body(tt_ref, w_ref, o_ref):
    parts = []
    for q in range(8):
        sub = tt_ref[:, q * _PROJ_SUB:(q + 1) * _PROJ_SUB]
        parts.append(lax.dot_general(
            sub, w_ref[...],
            dimension_numbers=(((0,), (1,)), ((), ())),
            preferred_element_type=jnp.float32,
        ))
    packed = jnp.concatenate(parts, axis=1)
    o_ref[...] = packed.reshape(_PROJ_SUB * 8 * LANES)


def _project(table_t, w16):
    return pl.pallas_call(
        _proj_body,
        grid=(_N_BLOCKS,),
        in_specs=[
            pl.BlockSpec((DIM, _PROJ_BLK), lambda i: (0, i)),
            pl.BlockSpec((LANES, DIM), lambda i: (0, 0)),
        ],
        out_specs=pl.BlockSpec((_PROJ_SUB * 8 * LANES,), lambda i: (i,)),
        out_shape=jax.ShapeDtypeStruct((_P_ROWS * 8 * LANES,), jnp.float32),
    )(table_t, w16)


IDX_PER_W = ROWS_PER_W * L
STREAM_IDX = 1600
N_STREAMS = IDX_PER_W // STREAM_IDX
ROWS_PER_STREAM = STREAM_IDX // L


def _pool_body(idx_hbm, p_hbm, bias_hbm, out_hbm, idx_v, rows_v, res_v, bias_v,
               sem0, sem1):
    wid = lax.axis_index("s") * NC + lax.axis_index("c")
    pltpu.sync_copy(idx_hbm.at[wid], idx_v)
    pltpu.sync_copy(bias_hbm, bias_v)
    inv_l = jnp.float32(1.0 / L)
    bias = bias_v[...]
    sems = (sem0, sem1)

    def fire(g):
        buf = g % 2
        return pltpu.async_copy(
            p_hbm.at[idx_v.at[pl.ds(g * STREAM_IDX, STREAM_IDX)]],
            rows_v.at[buf], sems[buf])

    pend = fire(0)
    for g in range(N_STREAMS):
        nxt = fire(g + 1) if g + 1 < N_STREAMS else None
        pend.wait()
        buf = g % 2

        def row_body(bloc, _):
            def red(i, a):
                a0, a1, a2, a3 = a
                base = bloc * L + 4 * i
                a0 = a0 + rows_v[buf, base, :]
                a1 = a1 + rows_v[buf, base + 1, :]
                a2 = a2 + rows_v[buf, base + 2, :]
                a3 = a3 + rows_v[buf, base + 3, :]
                return (a0, a1, a2, a3)

            z = jnp.zeros((LANES,), jnp.float32)
            a0, a1, a2, a3 = lax.fori_loop(0, L // 4, red, (z, z, z, z))
            total = (a0 + a1) + (a2 + a3)
            res_v[g * ROWS_PER_STREAM + bloc, :] = total * inv_l + bias
            return 0

        lax.fori_loop(0, ROWS_PER_STREAM, row_body, 0)
        pend = nxt
    pltpu.sync_copy(res_v, out_hbm.at[pl.ds(wid * ROWS_PER_W, ROWS_PER_W)])


@functools.cache
def _pool():
    return pl.kernel(
        _pool_body,
        out_type=jax.ShapeDtypeStruct((B, LANES), jnp.float32),
        mesh=plsc.VectorSubcoreMesh(
            core_axis_name="c", subcore_axis_name="s",
            num_cores=NC, num_subcores=NS,
        ),
        scratch_types=[
            pltpu.VMEM((IDX_PER_W,), jnp.int32),
            pltpu.VMEM((2, STREAM_IDX, LANES), jnp.float32),
            pltpu.VMEM((ROWS_PER_W, LANES), jnp.float32),
            pltpu.VMEM((LANES,), jnp.float32),
            pltpu.SemaphoreType.DMA,
            pltpu.SemaphoreType.DMA,
        ],
        compiler_params=pltpu.CompilerParams(use_tc_tiling_on_sc=False),
    )


@jax.jit
def kernel(text, table, W, b):
    w16 = jnp.zeros((LANES, DIM), jnp.float32).at[:OUT].set(W)
    bias16 = jnp.zeros((LANES,), jnp.float32).at[:OUT].set(b)
    p = _project(table.T, w16).reshape(_P_CHUNKS, LANES)
    v = text.astype(jnp.int32)
    remap = (v & -8192) | ((v & 1023) << 3) | ((v >> 10) & 7)
    idx = remap.reshape(NW, IDX_PER_W)
    pooled = _pool()(idx, p, bias16)
    return pooled[:, :OUT]
